# baseline (speedup 1.0000x reference)
import functools

import jax
import jax.numpy as jnp
from jax import lax
from jax.experimental import pallas as pl
from jax.experimental.pallas import tpu as pltpu
from jax.experimental.pallas import tpu_sc as plsc

NC = 2
NS = 16
NW = NC * NS
L = 16

B = 16384
D = 16
V = 1000000
BPW = B // NW
IDXW = 128
NCHUNK = BPW // IDXW


def _sc_body(uids_hbm, iids_hbm, utab_hbm, itab_hbm, out_hbm,
             uidx_v, iidx_v, udata_v, idata_v, out_v, sem):
    wid = lax.axis_index("s") * NC + lax.axis_index("c")

    pltpu.sync_copy(uids_hbm.at[wid], uidx_v)
    pltpu.sync_copy(iids_hbm.at[wid], iidx_v)

    copies = []
    for d in range(D):
        for j in range(NCHUNK):
            copies.append(pltpu.async_copy(
                utab_hbm.at[d].at[uidx_v.at[j]],
                udata_v.at[d, pl.ds(j * IDXW, IDXW)], sem))
            copies.append(pltpu.async_copy(
                itab_hbm.at[d].at[iidx_v.at[j]],
                idata_v.at[d, pl.ds(j * IDXW, IDXW)], sem))
    for cp in copies:
        cp.wait()

    def blk(k, carry):
        sl = pl.ds(k * L, L)
        acc = udata_v[0, sl] * idata_v[0, sl]
        for d in range(1, D):
            acc = acc + udata_v[d, sl] * idata_v[d, sl]
        out_v[sl] = acc
        return carry

    lax.fori_loop(0, BPW // L, blk, 0)
    pltpu.sync_copy(out_v, out_hbm.at[wid])


def kernel(user_ids, item_ids, user_table, item_table):
    mesh = plsc.VectorSubcoreMesh(core_axis_name="c", subcore_axis_name="s")

    sc_call = functools.partial(
        pl.kernel,
        out_type=jax.ShapeDtypeStruct((NW, BPW), jnp.float32),
        mesh=mesh,
        scratch_types=[
            pltpu.VMEM((NCHUNK, IDXW), jnp.int32),
            pltpu.VMEM((NCHUNK, IDXW), jnp.int32),
            pltpu.VMEM((D, BPW), jnp.float32),
            pltpu.VMEM((D, BPW), jnp.float32),
            pltpu.VMEM((BPW,), jnp.float32),
            pltpu.SemaphoreType.DMA,
        ],
        compiler_params=pltpu.CompilerParams(
            needs_layout_passes=False, use_tc_tiling_on_sc=False),
    )(_sc_body)

    uids = user_ids.astype(jnp.int32).reshape(NW, NCHUNK, IDXW)
    iids = item_ids.astype(jnp.int32).reshape(NW, NCHUNK, IDXW)
    out = sc_call(uids, iids, user_table.T, item_table.T)
    return out.reshape(B)
